# Initial kernel scaffold; baseline (speedup 1.0000x reference)
import jax
import jax.numpy as jnp
from jax.experimental import pallas as pl


def kernel(x, W):
    raise NotImplementedError("write your pallas kernel here")


# speedup vs baseline: 1.1105x; 1.1105x over previous
import functools

import jax
import jax.numpy as jnp
from jax import lax
from jax.experimental import pallas as pl
from jax.experimental.pallas import tpu as pltpu
from jax.experimental.pallas import tpu_sc as plsc

N_ROWS = 1_000_000
EMBED_DIM = 32
BATCH = 16384
HIST = 50
B_TOTAL = BATCH * HIST

NUM_CORES = 2
NUM_SUBCORES = 16
NUM_WORKERS = NUM_CORES * NUM_SUBCORES
B_PER_W = B_TOTAL // NUM_WORKERS
CHUNK = 3200
N_CHUNKS = B_PER_W // CHUNK


def _make_gather():
    mesh = plsc.VectorSubcoreMesh(core_axis_name="c", subcore_axis_name="s")

    @functools.partial(
        pl.kernel,
        mesh=mesh,
        out_type=jax.ShapeDtypeStruct((B_TOTAL, EMBED_DIM), jnp.float32),
        scratch_types=[
            pltpu.VMEM((CHUNK,), jnp.int32),
            pltpu.VMEM((CHUNK, EMBED_DIM), jnp.float32),
            pltpu.SemaphoreType.DMA,
        ],
        compiler_params=pltpu.CompilerParams(use_tc_tiling_on_sc=False),
    )
    def gather_kernel(idx_hbm, table_hbm, out_hbm, idx_v, rows_v, sem):
        wid = lax.axis_index("s") * NUM_CORES + lax.axis_index("c")
        for j in range(N_CHUNKS):
            base = wid * B_PER_W + j * CHUNK
            pltpu.sync_copy(idx_hbm.at[pl.ds(base, CHUNK)], idx_v)
            pltpu.async_copy(table_hbm.at[idx_v], rows_v, sem).wait()
            pltpu.sync_copy(rows_v, out_hbm.at[pl.ds(base, CHUNK)])

    return gather_kernel


_gather = _make_gather()


def kernel(x, W):
    idx = x.reshape(B_TOTAL).astype(jnp.int32)
    out = _gather(idx, W)
    return out.reshape(BATCH, HIST, EMBED_DIM)


# speedup vs baseline: 1.1129x; 1.0021x over previous
import functools

import jax
import jax.numpy as jnp
from jax import lax
from jax.experimental import pallas as pl
from jax.experimental.pallas import tpu as pltpu
from jax.experimental.pallas import tpu_sc as plsc

N_ROWS = 1_000_000
EMBED_DIM = 32
BATCH = 16384
HIST = 50
B_TOTAL = BATCH * HIST

NUM_CORES = 2
NUM_SUBCORES = 16
NUM_WORKERS = NUM_CORES * NUM_SUBCORES
B_PER_W = B_TOTAL // NUM_WORKERS
CHUNK = 1600
N_CHUNKS = B_PER_W // CHUNK


def _make_gather():
    mesh = plsc.VectorSubcoreMesh(core_axis_name="c", subcore_axis_name="s")

    @functools.partial(
        pl.kernel,
        mesh=mesh,
        out_type=jax.ShapeDtypeStruct((B_TOTAL, EMBED_DIM), jnp.float32),
        scratch_types=[
            pltpu.VMEM((B_PER_W,), jnp.int32),
            pltpu.VMEM((CHUNK, EMBED_DIM), jnp.float32),
            pltpu.VMEM((CHUNK, EMBED_DIM), jnp.float32),
            pltpu.SemaphoreType.DMA,
            pltpu.SemaphoreType.DMA,
            pltpu.SemaphoreType.DMA,
            pltpu.SemaphoreType.DMA,
        ],
        compiler_params=pltpu.CompilerParams(use_tc_tiling_on_sc=False),
    )
    def gather_kernel(idx_hbm, table_hbm, out_hbm, idx_v, rows0, rows1,
                      g0, g1, w0, w1):
        wid = lax.axis_index("s") * NUM_CORES + lax.axis_index("c")
        wbase = wid * B_PER_W
        rows = (rows0, rows1)
        gsem = (g0, g1)
        wsem = (w0, w1)

        pltpu.sync_copy(idx_hbm.at[pl.ds(wbase, B_PER_W)], idx_v)

        def gather_start(j):
            b = j % 2
            return pltpu.async_copy(
                table_hbm.at[idx_v.at[pl.ds(j * CHUNK, CHUNK)]],
                rows[b], gsem[b])

        def write_start(j):
            b = j % 2
            return pltpu.async_copy(
                rows[b], out_hbm.at[pl.ds(wbase + j * CHUNK, CHUNK)],
                wsem[b])

        gathers = [gather_start(0), gather_start(1)]
        writes = [None, None]
        for j in range(N_CHUNKS):
            b = j % 2
            gathers[b].wait()
            writes[b] = write_start(j)
            if j + 2 < N_CHUNKS:
                writes[b].wait()
                gathers[b] = gather_start(j + 2)
        writes[0].wait()
        writes[1].wait()

    return gather_kernel


_gather = _make_gather()


def kernel(x, W):
    idx = x.reshape(B_TOTAL).astype(jnp.int32)
    out = _gather(idx, W)
    return out.reshape(BATCH, HIST, EMBED_DIM)


# speedup vs baseline: 1.3591x; 1.2212x over previous
import functools

import jax
import jax.numpy as jnp
from jax import lax
from jax.experimental import pallas as pl
from jax.experimental.pallas import tpu as pltpu
from jax.experimental.pallas import tpu_sc as plsc

N_ROWS = 1_000_000
EMBED_DIM = 32
BATCH = 16384
HIST = 50

NUM_CORES = 2
NUM_SUBCORES = 16
NUM_WORKERS = NUM_CORES * NUM_SUBCORES
B_PER_W = BATCH // NUM_WORKERS
LANES = 16


def _make_gather():
    mesh = plsc.VectorSubcoreMesh(core_axis_name="c", subcore_axis_name="s")

    @functools.partial(
        pl.kernel,
        mesh=mesh,
        out_type=jax.ShapeDtypeStruct((HIST, EMBED_DIM, BATCH), jnp.float32),
        scratch_types=[
            pltpu.VMEM((B_PER_W,), jnp.int32),
            pltpu.VMEM((B_PER_W, EMBED_DIM), jnp.float32),
            pltpu.VMEM((EMBED_DIM, B_PER_W), jnp.float32),
            pltpu.SemaphoreType.DMA,
        ],
        compiler_params=pltpu.CompilerParams(
            use_tc_tiling_on_sc=False, needs_layout_passes=False),
    )
    def gather_kernel(xt_hbm, table_hbm, out_hbm, idx_v, rows_v, trans_v, gsem):
        wid = lax.axis_index("s") * NUM_CORES + lax.axis_index("c")
        b0 = wid * B_PER_W
        lane = lax.broadcasted_iota(jnp.int32, (LANES,), 0)

        def h_body(h, carry):
            pltpu.sync_copy(xt_hbm.at[h, pl.ds(b0, B_PER_W)], idx_v)
            pltpu.async_copy(table_hbm.at[idx_v], rows_v, gsem).wait()

            def c_body(c, carry2):
                col = jnp.full((LANES,), c, jnp.int32)
                for l0 in range(0, B_PER_W, LANES):
                    v = plsc.load_gather(rows_v, [lane + l0, col])
                    trans_v[c, pl.ds(l0, LANES)] = v
                return carry2

            lax.fori_loop(0, EMBED_DIM, c_body, 0, unroll=False)
            pltpu.sync_copy(trans_v, out_hbm.at[h, :, pl.ds(b0, B_PER_W)])
            return carry

        lax.fori_loop(0, HIST, h_body, 0, unroll=False)

    return gather_kernel


_gather = _make_gather()


def kernel(x, W):
    xt = x.T.astype(jnp.int32)
    out_t = _gather(xt, W)
    return lax.transpose(out_t, (2, 0, 1))


# speedup vs baseline: 1.4563x; 1.0715x over previous
import functools

import jax
import jax.numpy as jnp
from jax import lax
from jax.experimental import pallas as pl
from jax.experimental.pallas import tpu as pltpu
from jax.experimental.pallas import tpu_sc as plsc

N_ROWS = 1_000_000
EMBED_DIM = 32
BATCH = 16384
HIST = 50

NUM_CORES = 2
NUM_SUBCORES = 16
LANES = 16

H_GROUPS = 2
B_GROUPS = 16
H_PER_W = HIST // H_GROUPS
B_PER_W = BATCH // B_GROUPS


def _make_gather():
    mesh = plsc.VectorSubcoreMesh(core_axis_name="c", subcore_axis_name="s")

    @functools.partial(
        pl.kernel,
        mesh=mesh,
        out_type=jax.ShapeDtypeStruct((HIST, EMBED_DIM, BATCH), jnp.float32),
        scratch_types=[
            pltpu.VMEM((H_PER_W, B_PER_W), jnp.int32),
            pltpu.VMEM((B_PER_W, EMBED_DIM), jnp.float32),
            pltpu.VMEM((B_PER_W, EMBED_DIM), jnp.float32),
            pltpu.VMEM((EMBED_DIM, B_PER_W), jnp.float32),
            pltpu.SemaphoreType.DMA,
            pltpu.SemaphoreType.DMA,
            pltpu.SemaphoreType.DMA,
        ],
        compiler_params=pltpu.CompilerParams(
            use_tc_tiling_on_sc=False, needs_layout_passes=False),
    )
    def gather_kernel(xt_hbm, table_hbm, out_hbm, idx_v, rows0, rows1,
                      trans_v, g0, g1, wsem):
        wid = lax.axis_index("s") * NUM_CORES + lax.axis_index("c")
        hg = wid // B_GROUPS
        h0 = hg * H_PER_W
        bg = wid % B_GROUPS
        b0 = bg * B_PER_W
        rows = (rows0, rows1)
        gsem = (g0, g1)
        lane = lax.broadcasted_iota(jnp.int32, (LANES,), 0)

        pltpu.sync_copy(
            xt_hbm.at[pl.ds(h0, H_PER_W), pl.ds(b0, B_PER_W)], idx_v)

        def gstart(j):
            b = j % 2
            return pltpu.async_copy(table_hbm.at[idx_v.at[j]], rows[b],
                                    gsem[b])

        gathers = [gstart(0), None]
        wcopy = None
        for j in range(H_PER_W):
            b = j % 2
            gathers[b].wait()
            if j + 1 < H_PER_W:
                gathers[1 - b] = gstart(j + 1)
            if wcopy is not None:
                wcopy.wait()

            def l_body(l0, carry):
                row_idx = lane + l0 * LANES
                for c in range(EMBED_DIM):
                    v = plsc.load_gather(
                        rows[b], [row_idx, jnp.full((LANES,), c, jnp.int32)])
                    trans_v[c, pl.ds(l0 * LANES, LANES)] = v
                return carry

            lax.fori_loop(0, B_PER_W // LANES, l_body, 0, unroll=False)
            wcopy = pltpu.async_copy(
                trans_v, out_hbm.at[h0 + j, :, pl.ds(b0, B_PER_W)], wsem)
        wcopy.wait()

    return gather_kernel


_gather = _make_gather()


def kernel(x, W):
    xt = x.T.astype(jnp.int32)
    out_t = _gather(xt, W)
    return lax.transpose(out_t, (2, 0, 1))


# speedup vs baseline: 2.2326x; 1.5331x over previous
import functools

import jax
import jax.numpy as jnp
from jax import lax
from jax.experimental import pallas as pl
from jax.experimental.pallas import tpu as pltpu
from jax.experimental.pallas import tpu_sc as plsc

N_ROWS = 1_000_000
EMBED_DIM = 32
BATCH = 16384
HIST = 50

NUM_CORES = 2
NUM_SUBCORES = 16
LANES = 16

H_GROUPS = 2
B_GROUPS = 16
H_PER_W = HIST // H_GROUPS
B_PER_W = BATCH // B_GROUPS


def _make_gather():
    mesh = plsc.VectorSubcoreMesh(core_axis_name="c", subcore_axis_name="s")

    @functools.partial(
        pl.kernel,
        mesh=mesh,
        out_type=jax.ShapeDtypeStruct((HIST, EMBED_DIM, BATCH), jnp.float32),
        scratch_types=[
            pltpu.VMEM((H_PER_W, B_PER_W), jnp.int32),
            pltpu.VMEM((B_PER_W, EMBED_DIM), jnp.float32),
            pltpu.VMEM((B_PER_W, EMBED_DIM), jnp.float32),
            pltpu.VMEM((EMBED_DIM, B_PER_W + 1), jnp.float32),
            pltpu.SemaphoreType.DMA,
            pltpu.SemaphoreType.DMA,
        ],
        compiler_params=pltpu.CompilerParams(
            use_tc_tiling_on_sc=False, needs_layout_passes=False,
            disable_bounds_checks=True),
    )
    def gather_kernel(xt_hbm, table_hbm, out_hbm, idx_v, rows0, rows1,
                      trans_v, g0, g1):
        wid = lax.axis_index("s") * NUM_CORES + lax.axis_index("c")
        hg = wid // B_GROUPS
        h0 = hg * H_PER_W
        bg = wid % B_GROUPS
        b0 = bg * B_PER_W
        rows = (rows0, rows1)
        gsem = (g0, g1)
        lane = lax.broadcasted_iota(jnp.int32, (LANES,), 0)
        SUB = 16

        pltpu.sync_copy(
            xt_hbm.at[pl.ds(h0, H_PER_W), pl.ds(b0, B_PER_W)], idx_v)

        def gstart(j, b):
            pltpu.async_copy(table_hbm.at[idx_v.at[j]], rows[b], gsem[b])

        def gwait(j, b):
            pltpu.make_async_copy(table_hbm.at[idx_v.at[j]], rows[b],
                                  gsem[b]).wait()

        def transpose_and_store(j, b):
            def b_body(bb, carry):
                for s in range(SUB):
                    brow = bb * SUB + s
                    col = jnp.full((LANES,), brow, jnp.int32)
                    for c0 in (0, LANES):
                        v = rows[b][brow, pl.ds(c0, LANES)]
                        plsc.store_scatter(trans_v, [lane + c0, col], v)
                return carry

            lax.fori_loop(0, B_PER_W // SUB, b_body, 0, unroll=False)
            pltpu.sync_copy(
                trans_v.at[:, pl.ds(0, B_PER_W)],
                out_hbm.at[h0 + j, :, pl.ds(b0, B_PER_W)])

        gstart(0, 0)

        def pair_body(j2, carry):
            j = 2 * j2
            gwait(j, 0)
            gstart(j + 1, 1)
            transpose_and_store(j, 0)
            gwait(j + 1, 1)
            gstart(j + 2, 0)
            transpose_and_store(j + 1, 1)
            return carry

        lax.fori_loop(0, (H_PER_W - 1) // 2, pair_body, 0, unroll=False)
        gwait(H_PER_W - 1, 0)
        transpose_and_store(H_PER_W - 1, 0)

    return gather_kernel


_gather = _make_gather()


def kernel(x, W):
    xt = x.T.astype(jnp.int32)
    out_t = _gather(xt, W)
    return lax.transpose(out_t, (2, 0, 1))
